# baseline (speedup 1.0000x reference)
import functools
import math

import jax
import jax.numpy as jnp
from jax import lax
from jax.experimental import pallas as pl
from jax.experimental.pallas import tpu as pltpu
from jax.experimental.pallas import tpu_sc as plsc

_VOCAB = 1000000
_HIDDEN = 64
_SCALE = math.sqrt(_HIDDEN)

_B, _T = 16384, 50
_TOTAL = _B * _T
_IDXW = 128
_ROWS = _TOTAL // _IDXW
_NW = 32
_ROWS_PER_W = _ROWS // _NW
_GRP = 8
_NGRP = _ROWS_PER_W // _GRP

_mesh = plsc.VectorSubcoreMesh(core_axis_name="c", subcore_axis_name="s")


@functools.partial(
    pl.kernel,
    mesh=_mesh,
    out_type=jax.ShapeDtypeStruct((_ROWS, _IDXW, _HIDDEN), jnp.float32),
    scratch_types=[
        pltpu.VMEM((_GRP, _IDXW), jnp.int32),
        pltpu.VMEM((_GRP, _IDXW, _HIDDEN), jnp.float32),
        pltpu.SemaphoreType.DMA,
    ],
    compiler_params=pltpu.CompilerParams(use_tc_tiling_on_sc=False),
)
def _embed(ids_hbm, table_hbm, out_hbm, idx_v, rows_v, sem):
    nc = 2
    wid = lax.axis_index("s") * nc + lax.axis_index("c")
    base = wid * _ROWS_PER_W

    def group_body(g, _):
        row_off = base + g * _GRP
        pltpu.sync_copy(ids_hbm.at[pl.ds(row_off, _GRP)], idx_v)
        copies = [
            pltpu.async_copy(table_hbm.at[idx_v.at[j]], rows_v.at[j], sem)
            for j in range(_GRP)
        ]
        for c in copies:
            c.wait()

        def scale_body(i, _):
            for j in range(_GRP):
                for k in range(_HIDDEN // 16):
                    sl = pl.ds(k * 16, 16)
                    rows_v[j, i, sl] = rows_v[j, i, sl] * _SCALE
            return 0

        lax.fori_loop(0, _IDXW, scale_body, 0)
        pltpu.sync_copy(rows_v, out_hbm.at[pl.ds(row_off, _GRP)])
        return 0

    lax.fori_loop(0, _NGRP, group_body, 0)


def kernel(input_ids, embed_weight):
    ids = input_ids.astype(jnp.int32).reshape(_ROWS, _IDXW)
    out = _embed(ids, embed_weight)
    return out.reshape(_B, _T, _HIDDEN)
